# baseline (speedup 1.0000x reference)
import jax
import jax.numpy as jnp
from jax.experimental import pallas as pl
from jax.experimental.pallas import tpu as pltpu

_TN = 2048
_NBUF = 4
_NSPLIT = 2


def _mlp_body(idx_ref, emb_ref, w1_ref, b1_ref, b2_ref, w2_hbm, out_hbm,
              htab_ref, oh_ref, w2buf, outbuf, rsem, wsem):
    T, V = idx_ref.shape[0], emb_ref.shape[0]
    N = w2_hbm.shape[1]
    steps = N // _TN

    H = w2_hbm.shape[0]
    HS = H // _NSPLIT

    def read_start(j, slot):
        for r in range(_NSPLIT):
            pltpu.make_async_copy(
                w2_hbm.at[pl.ds(r * HS, HS), pl.ds(j * _TN, _TN)],
                w2buf.at[slot, pl.ds(r * HS, HS)],
                rsem.at[slot, r],
            ).start()

    def read_wait(j, slot):
        for r in range(_NSPLIT):
            pltpu.make_async_copy(
                w2_hbm.at[pl.ds(r * HS, HS), pl.ds(j * _TN, _TN)],
                w2buf.at[slot, pl.ds(r * HS, HS)],
                rsem.at[slot, r],
            ).wait()

    for p in range(_NBUF):
        read_start(p, p)

    h = jnp.dot(emb_ref[...], w1_ref[...], preferred_element_type=jnp.float32)
    htab_ref[...] = jnp.tanh(h + b1_ref[...])
    iota = jax.lax.broadcasted_iota(jnp.int32, (T, V), 1)
    oh_ref[...] = jnp.where(iota == idx_ref[...], 1.0, 0.0).astype(jnp.float32)

    for j in range(steps):
        slot = j % _NBUF
        read_wait(j, slot)
        m = jnp.dot(htab_ref[...], w2buf[slot], preferred_element_type=jnp.float32)
        o = (
            jnp.dot(oh_ref[...], m, preferred_element_type=jnp.float32)
            + b2_ref[:, j * _TN:(j + 1) * _TN]
        )
        if j >= _NBUF:
            pltpu.make_async_copy(
                outbuf.at[slot],
                out_hbm.at[:, pl.ds((j - _NBUF) * _TN, _TN)],
                wsem.at[slot],
            ).wait()
        outbuf[slot] = o
        pltpu.make_async_copy(
            outbuf.at[slot], out_hbm.at[:, pl.ds(j * _TN, _TN)], wsem.at[slot]
        ).start()
        if j + _NBUF < steps:
            read_start(j + _NBUF, slot)

    for j in range(max(0, steps - _NBUF), steps):
        slot = j % _NBUF
        pltpu.make_async_copy(
            outbuf.at[slot], out_hbm.at[:, pl.ds(j * _TN, _TN)], wsem.at[slot]
        ).wait()


def kernel(prefix, emb, W1, b1, W2, b2):
    B, L = prefix.shape
    V, D = emb.shape
    H = W1.shape[1]
    N = W2.shape[1]
    T = B * L

    idx = prefix.reshape(T, 1).astype(jnp.int32)

    out = pl.pallas_call(
        _mlp_body,
        in_specs=[
            pl.BlockSpec(memory_space=pltpu.MemorySpace.VMEM),
            pl.BlockSpec(memory_space=pltpu.MemorySpace.VMEM),
            pl.BlockSpec(memory_space=pltpu.MemorySpace.VMEM),
            pl.BlockSpec(memory_space=pltpu.MemorySpace.VMEM),
            pl.BlockSpec(memory_space=pltpu.MemorySpace.VMEM),
            pl.BlockSpec(memory_space=pl.ANY),
        ],
        out_specs=pl.BlockSpec(memory_space=pl.ANY),
        out_shape=jax.ShapeDtypeStruct((T, N), jnp.float32),
        scratch_shapes=[
            pltpu.VMEM((V, H), jnp.float32),
            pltpu.VMEM((T, V), jnp.float32),
            pltpu.VMEM((_NBUF, H, _TN), jnp.float32),
            pltpu.VMEM((_NBUF, T, _TN), jnp.float32),
            pltpu.SemaphoreType.DMA((_NBUF, _NSPLIT)),
            pltpu.SemaphoreType.DMA((_NBUF,)),
        ],
    )(idx, emb, W1, b1.reshape(1, H), b2.reshape(1, N), W2)

    return out.reshape(B, L, N)
